# baseline (speedup 1.0000x reference)
import jax
import jax.numpy as jnp
import numpy as np
from jax.experimental import pallas as pl
from jax.experimental.pallas import tpu as pltpu

N = 2048
E = 4096
D_NODE = 128
D_EDGE = 16

BN = 512
BE = 512
GN = N // BN
GE = E // BE

_NODE_PAIRS = [(i, j) for i in range(GN) for j in range(i, GN)]
_EDGE_PAIRS = [(i, j) for i in range(GE) for j in range(i, GE)]
NP_STEPS = len(_NODE_PAIRS)
EP_STEPS = len(_EDGE_PAIRS)
T_STEPS = NP_STEPS + EP_STEPS

_F32 = jnp.float32
_BF16 = jnp.bfloat16


def _build_table() -> np.ndarray:
    tab = np.zeros((8, T_STEPS), dtype=np.int32)
    lv2_prev = (0, 0)
    for t, (i, j) in enumerate(_NODE_PAIRS):
        tab[0, t], tab[1, t] = i, j
        if i != j:
            lv2_prev = (j, i)
        tab[2, t], tab[3, t] = lv2_prev
    tab[0, NP_STEPS:], tab[1, NP_STEPS:] = _NODE_PAIRS[-1]
    tab[2, NP_STEPS:], tab[3, NP_STEPS:] = lv2_prev
    le2_prev = (0, 0)
    for s, (i, j) in enumerate(_EDGE_PAIRS):
        t = NP_STEPS + s
        tab[4, t], tab[5, t] = i, j
        if i != j:
            le2_prev = (j, i)
        tab[6, t], tab[7, t] = le2_prev
    return tab


def _mega_kernel(tab_ref, inc_ref, lv1_ref, lv2_ref, le1_ref, le2_ref,
                 x_ref, e_ref, wn_ref, we_ref, pn_ref, pe_ref,
                 bn_ref, be_ref,
                 nodes_ref, edges_ref,
                 u_ref, phiv_ref, xw_ref, ew_ref, nacc_ref, eacc_ref):
    t = pl.program_id(0)

    @pl.when(t == 0)
    def _():
        phi_e = jax.lax.dot_general(
            pn_ref[...], e_ref[...], (((0,), (1,)), ((), ())),
            preferred_element_type=_F32).astype(_BF16)
        u_ref[...] = inc_ref[...] * phi_e
        phiv_ref[...] = jnp.dot(x_ref[...], pe_ref[...],
                                preferred_element_type=_F32).astype(_BF16)
        xw_ref[...] = jnp.dot(x_ref[...], wn_ref[...],
                              preferred_element_type=_F32).astype(_BF16)
        ew_ref[...] = jnp.dot(e_ref[...], we_ref[...],
                              preferred_element_type=_F32).astype(_BF16)
        nacc_ref[...] = jnp.zeros_like(nacc_ref)
        eacc_ref[...] = jnp.zeros_like(eacc_ref)

    @pl.when(t < NP_STEPS)
    def _():
        i = tab_ref[0, t]
        j = tab_ref[1, t]
        a = u_ref[pl.ds(i * BN, BN), :]
        b = inc_ref[pl.ds(j * BN, BN), :]
        p = jax.lax.dot_general(a, b, (((1,), (1,)), ((), ())),
                                preferred_element_type=_F32)
        ci = jnp.dot((p * lv1_ref[...]).astype(_BF16),
                     xw_ref[pl.ds(j * BN, BN), :],
                     preferred_element_type=_F32)
        nacc_ref[pl.ds(i * BN, BN), :] += ci

        @pl.when(i != j)
        def _():
            pt = jax.lax.transpose(p.astype(_BF16), (1, 0))
            cj = jnp.dot((pt * lv2_ref[...]).astype(_BF16),
                         xw_ref[pl.ds(i * BN, BN), :],
                         preferred_element_type=_F32)
            nacc_ref[pl.ds(j * BN, BN), :] += cj

    @pl.when(t >= NP_STEPS)
    def _():
        i = tab_ref[4, t]
        j = tab_ref[5, t]
        ci_op = inc_ref[:, pl.ds(i * BE, BE)]
        d = inc_ref[:, pl.ds(j * BE, BE)] * phiv_ref[...]
        p = jax.lax.dot_general(ci_op, d, (((0,), (0,)), ((), ())),
                                preferred_element_type=_F32)
        ci = jnp.dot((p * le1_ref[...]).astype(_BF16),
                     ew_ref[pl.ds(j * BE, BE), :],
                     preferred_element_type=_F32)
        eacc_ref[pl.ds(i * BE, BE), :] += ci

        @pl.when(i != j)
        def _():
            pt = jax.lax.transpose(p.astype(_BF16), (1, 0))
            cj = jnp.dot((pt * le2_ref[...]).astype(_BF16),
                         ew_ref[pl.ds(i * BE, BE), :],
                         preferred_element_type=_F32)
            eacc_ref[pl.ds(j * BE, BE), :] += cj

    @pl.when(t == T_STEPS - 1)
    def _():
        nodes_ref[...] = nacc_ref[...] + x_ref[...] + bn_ref[...]
        edges_ref[...] = eacc_ref[...] + e_ref[...] + be_ref[...]


def kernel(x, node_laplacian, edge_laplacian, incidence, e, W_n, W_e,
           p_node, p_edge, b_n, b_e):
    bn2 = b_n.reshape(1, D_NODE)
    be2 = b_e.reshape(1, D_EDGE)
    inc_bf = incidence.astype(_BF16)
    tab = jnp.asarray(_build_table())

    full = lambda t, tab_ref: (0, 0)

    grid_spec = pltpu.PrefetchScalarGridSpec(
        num_scalar_prefetch=1,
        grid=(T_STEPS,),
        in_specs=[
            pl.BlockSpec((N, E), full),
            pl.BlockSpec((BN, BN), lambda t, s: (s[0, t], s[1, t])),
            pl.BlockSpec((BN, BN), lambda t, s: (s[2, t], s[3, t])),
            pl.BlockSpec((BE, BE), lambda t, s: (s[4, t], s[5, t])),
            pl.BlockSpec((BE, BE), lambda t, s: (s[6, t], s[7, t])),
            pl.BlockSpec((N, D_NODE), full),
            pl.BlockSpec((E, D_EDGE), full),
            pl.BlockSpec((D_NODE, D_NODE), full),
            pl.BlockSpec((D_EDGE, D_EDGE), full),
            pl.BlockSpec((D_EDGE, 1), full),
            pl.BlockSpec((D_NODE, 1), full),
            pl.BlockSpec((1, D_NODE), full),
            pl.BlockSpec((1, D_EDGE), full),
        ],
        out_specs=[
            pl.BlockSpec((N, D_NODE), full),
            pl.BlockSpec((E, D_EDGE), full),
        ],
        scratch_shapes=[
            pltpu.VMEM((N, E), _BF16),
            pltpu.VMEM((N, 1), _BF16),
            pltpu.VMEM((N, D_NODE), _BF16),
            pltpu.VMEM((E, D_EDGE), _BF16),
            pltpu.VMEM((N, D_NODE), _F32),
            pltpu.VMEM((E, D_EDGE), _F32),
        ],
    )

    new_nodes, new_edges = pl.pallas_call(
        _mega_kernel,
        grid_spec=grid_spec,
        out_shape=[
            jax.ShapeDtypeStruct((N, D_NODE), _F32),
            jax.ShapeDtypeStruct((E, D_EDGE), _F32),
        ],
        compiler_params=pltpu.CompilerParams(
            dimension_semantics=("arbitrary",)),
    )(tab, inc_bf, node_laplacian, node_laplacian,
      edge_laplacian, edge_laplacian, x, e, W_n, W_e,
      p_node, p_edge, bn2, be2)

    return new_nodes, new_edges


# speedup vs baseline: 1.0219x; 1.0219x over previous
import jax
import jax.numpy as jnp
import numpy as np
from jax.experimental import pallas as pl
from jax.experimental.pallas import tpu as pltpu

N = 2048
E = 4096
D_NODE = 128
D_EDGE = 16

BN = 512
BE = 512
GN = N // BN
GE = E // BE

_NODE_PAIRS = [(i, j) for i in range(GN) for j in range(i, GN)]
_EDGE_PAIRS = [(i, j) for i in range(GE) for j in range(i, GE)]
NP_STEPS = len(_NODE_PAIRS)
EP_STEPS = len(_EDGE_PAIRS)
T_STEPS = NP_STEPS + EP_STEPS

_F32 = jnp.float32
_BF16 = jnp.bfloat16


def _build_table() -> np.ndarray:
    tab = np.zeros((8, T_STEPS), dtype=np.int32)
    lv2_prev = (0, 0)
    for t, (i, j) in enumerate(_NODE_PAIRS):
        tab[0, t], tab[1, t] = i, j
        if i != j:
            lv2_prev = (j, i)
        tab[2, t], tab[3, t] = lv2_prev
    tab[0, NP_STEPS:], tab[1, NP_STEPS:] = _NODE_PAIRS[-1]
    tab[2, NP_STEPS:], tab[3, NP_STEPS:] = lv2_prev
    le2_prev = (0, 0)
    for s, (i, j) in enumerate(_EDGE_PAIRS):
        t = NP_STEPS + s
        tab[4, t], tab[5, t] = i, j
        if i != j:
            le2_prev = (j, i)
        tab[6, t], tab[7, t] = le2_prev
    return tab


def _mega_kernel(tab_ref, inc_ref, lv1_ref, lv2_ref, le1_ref, le2_ref,
                 x_ref, e_ref, wn_ref, we_ref, pn_ref, pe_ref,
                 bn_ref, be_ref,
                 nodes_ref, edges_ref,
                 u_ref, phie_ref, phiv_ref, xw_ref, ew_ref,
                 nacc_ref, eacc_ref):
    t = pl.program_id(0)

    @pl.when(t == 0)
    def _():
        phie_ref[...] = jax.lax.dot_general(
            pn_ref[...], e_ref[...], (((0,), (1,)), ((), ())),
            preferred_element_type=_F32).astype(_BF16)
        phiv_ref[...] = jnp.dot(x_ref[...], pe_ref[...],
                                preferred_element_type=_F32).astype(_BF16)
        xw_ref[...] = jnp.dot(x_ref[...], wn_ref[...],
                              preferred_element_type=_F32).astype(_BF16)
        ew_ref[...] = jnp.dot(e_ref[...], we_ref[...],
                              preferred_element_type=_F32).astype(_BF16)
        nacc_ref[...] = jnp.zeros_like(nacc_ref)
        eacc_ref[...] = jnp.zeros_like(eacc_ref)

    @pl.when(t < NP_STEPS)
    def _():
        i = tab_ref[0, t]
        j = tab_ref[1, t]

        @pl.when(i == j)
        def _():
            u_ref[pl.ds(i * BN, BN), :] = (inc_ref[pl.ds(i * BN, BN), :]
                                           * phie_ref[...])

        a = u_ref[pl.ds(i * BN, BN), :]
        b = inc_ref[pl.ds(j * BN, BN), :]
        p = jax.lax.dot_general(a, b, (((1,), (1,)), ((), ())),
                                preferred_element_type=_F32)
        ci = jnp.dot((p * lv1_ref[...]).astype(_BF16),
                     xw_ref[pl.ds(j * BN, BN), :],
                     preferred_element_type=_F32)
        nacc_ref[pl.ds(i * BN, BN), :] += ci

        @pl.when(i != j)
        def _():
            pt = jax.lax.transpose(p.astype(_BF16), (1, 0))
            cj = jnp.dot((pt * lv2_ref[...]).astype(_BF16),
                         xw_ref[pl.ds(i * BN, BN), :],
                         preferred_element_type=_F32)
            nacc_ref[pl.ds(j * BN, BN), :] += cj

    @pl.when(t >= NP_STEPS)
    def _():
        i = tab_ref[4, t]
        j = tab_ref[5, t]
        ci_op = inc_ref[:, pl.ds(i * BE, BE)]
        d = inc_ref[:, pl.ds(j * BE, BE)] * phiv_ref[...]
        p = jax.lax.dot_general(ci_op, d, (((0,), (0,)), ((), ())),
                                preferred_element_type=_F32)
        ci = jnp.dot((p * le1_ref[...]).astype(_BF16),
                     ew_ref[pl.ds(j * BE, BE), :],
                     preferred_element_type=_F32)
        eacc_ref[pl.ds(i * BE, BE), :] += ci

        @pl.when(i != j)
        def _():
            pt = jax.lax.transpose(p.astype(_BF16), (1, 0))
            cj = jnp.dot((pt * le2_ref[...]).astype(_BF16),
                         ew_ref[pl.ds(i * BE, BE), :],
                         preferred_element_type=_F32)
            eacc_ref[pl.ds(j * BE, BE), :] += cj

    @pl.when(t == T_STEPS - 1)
    def _():
        nodes_ref[...] = nacc_ref[...] + x_ref[...] + bn_ref[...]
        edges_ref[...] = eacc_ref[...] + e_ref[...] + be_ref[...]


def kernel(x, node_laplacian, edge_laplacian, incidence, e, W_n, W_e,
           p_node, p_edge, b_n, b_e):
    bn2 = b_n.reshape(1, D_NODE)
    be2 = b_e.reshape(1, D_EDGE)
    inc_bf = incidence.astype(_BF16)
    tab = jnp.asarray(_build_table())

    full = lambda t, tab_ref: (0, 0)

    grid_spec = pltpu.PrefetchScalarGridSpec(
        num_scalar_prefetch=1,
        grid=(T_STEPS,),
        in_specs=[
            pl.BlockSpec((N, E), full),
            pl.BlockSpec((BN, BN), lambda t, s: (s[0, t], s[1, t])),
            pl.BlockSpec((BN, BN), lambda t, s: (s[2, t], s[3, t])),
            pl.BlockSpec((BE, BE), lambda t, s: (s[4, t], s[5, t])),
            pl.BlockSpec((BE, BE), lambda t, s: (s[6, t], s[7, t])),
            pl.BlockSpec((N, D_NODE), full),
            pl.BlockSpec((E, D_EDGE), full),
            pl.BlockSpec((D_NODE, D_NODE), full),
            pl.BlockSpec((D_EDGE, D_EDGE), full),
            pl.BlockSpec((D_EDGE, 1), full),
            pl.BlockSpec((D_NODE, 1), full),
            pl.BlockSpec((1, D_NODE), full),
            pl.BlockSpec((1, D_EDGE), full),
        ],
        out_specs=[
            pl.BlockSpec((N, D_NODE), full),
            pl.BlockSpec((E, D_EDGE), full),
        ],
        scratch_shapes=[
            pltpu.VMEM((N, E), _BF16),
            pltpu.VMEM((1, E), _BF16),
            pltpu.VMEM((N, 1), _BF16),
            pltpu.VMEM((N, D_NODE), _BF16),
            pltpu.VMEM((E, D_EDGE), _BF16),
            pltpu.VMEM((N, D_NODE), _F32),
            pltpu.VMEM((E, D_EDGE), _F32),
        ],
    )

    new_nodes, new_edges = pl.pallas_call(
        _mega_kernel,
        grid_spec=grid_spec,
        out_shape=[
            jax.ShapeDtypeStruct((N, D_NODE), _F32),
            jax.ShapeDtypeStruct((E, D_EDGE), _F32),
        ],
        compiler_params=pltpu.CompilerParams(
            dimension_semantics=("arbitrary",)),
    )(tab, inc_bf, node_laplacian, node_laplacian,
      edge_laplacian, edge_laplacian, x, e, W_n, W_e,
      p_node, p_edge, bn2, be2)

    return new_nodes, new_edges


# speedup vs baseline: 1.0277x; 1.0057x over previous
import jax
import jax.numpy as jnp
import numpy as np
from jax.experimental import pallas as pl
from jax.experimental.pallas import tpu as pltpu

N = 2048
E = 4096
D_NODE = 128
D_EDGE = 16

BN = 512
BE = 512
GN = N // BN
GE = E // BE

_NODE_PAIRS = [(i, j) for i in range(GN) for j in range(i, GN)]
_EDGE_PAIRS = [(i, j) for i in range(GE) for j in range(i, GE)]
NP_STEPS = len(_NODE_PAIRS)
EP_STEPS = len(_EDGE_PAIRS)
T_STEPS = NP_STEPS + EP_STEPS

_F32 = jnp.float32
_BF16 = jnp.bfloat16


def _build_table() -> np.ndarray:
    tab = np.zeros((8, T_STEPS), dtype=np.int32)
    lv2_prev = (0, 0)
    for t, (i, j) in enumerate(_NODE_PAIRS):
        tab[0, t], tab[1, t] = i, j
        if i != j:
            lv2_prev = (j, i)
        tab[2, t], tab[3, t] = lv2_prev
    tab[0, NP_STEPS:], tab[1, NP_STEPS:] = _NODE_PAIRS[-1]
    tab[2, NP_STEPS:], tab[3, NP_STEPS:] = lv2_prev
    le2_prev = (0, 0)
    for s, (i, j) in enumerate(_EDGE_PAIRS):
        t = NP_STEPS + s
        tab[4, t], tab[5, t] = i, j
        if i != j:
            le2_prev = (j, i)
        tab[6, t], tab[7, t] = le2_prev
    return tab


def _mega_kernel(tab_ref, inc_ref, lv1_ref, lv2_ref, le1_ref, le2_ref,
                 x_ref, e_ref, wn_ref, we_ref, pn_ref, pe_ref,
                 bn_ref, be_ref,
                 nodes_ref, edges_ref,
                 u_ref, phie_ref, phiv_ref, xw_ref, ew_ref,
                 vslab_ref, nacc_ref, eacc_ref):
    t = pl.program_id(0)

    @pl.when(t == 0)
    def _():
        phie_ref[...] = jax.lax.dot_general(
            pn_ref[...], e_ref[...], (((0,), (1,)), ((), ())),
            preferred_element_type=_F32).astype(_BF16)
        phiv_ref[...] = jnp.dot(x_ref[...], pe_ref[...],
                                preferred_element_type=_F32).astype(_BF16)
        xw_ref[...] = jnp.dot(x_ref[...], wn_ref[...],
                              preferred_element_type=_F32).astype(_BF16)
        ew_ref[...] = jnp.dot(e_ref[...], we_ref[...],
                              preferred_element_type=_F32).astype(_BF16)
        nacc_ref[...] = jnp.zeros_like(nacc_ref)
        eacc_ref[...] = jnp.zeros_like(eacc_ref)

    @pl.when(t < NP_STEPS)
    def _():
        i = tab_ref[0, t]
        j = tab_ref[1, t]

        @pl.when(i == j)
        def _():
            u_ref[pl.ds(i * BN, BN), :] = (inc_ref[pl.ds(i * BN, BN), :]
                                           * phie_ref[...])

        a = u_ref[pl.ds(i * BN, BN), :]
        b = inc_ref[pl.ds(j * BN, BN), :]
        p = jax.lax.dot_general(a, b, (((1,), (1,)), ((), ())),
                                preferred_element_type=_F32)
        ci = jnp.dot((p * lv1_ref[...]).astype(_BF16),
                     xw_ref[pl.ds(j * BN, BN), :],
                     preferred_element_type=_F32)
        nacc_ref[pl.ds(i * BN, BN), :] += ci

        @pl.when(i != j)
        def _():
            pt = jax.lax.transpose(p.astype(_BF16), (1, 0))
            cj = jnp.dot((pt * lv2_ref[...]).astype(_BF16),
                         xw_ref[pl.ds(i * BN, BN), :],
                         preferred_element_type=_F32)
            nacc_ref[pl.ds(j * BN, BN), :] += cj

    @pl.when(t >= NP_STEPS)
    def _():
        i = tab_ref[4, t]
        j = tab_ref[5, t]

        @pl.when(i == j)
        def _():
            vslab_ref[...] = inc_ref[:, pl.ds(i * BE, BE)] * phiv_ref[...]

        p = jax.lax.dot_general(vslab_ref[...],
                                inc_ref[:, pl.ds(j * BE, BE)],
                                (((0,), (0,)), ((), ())),
                                preferred_element_type=_F32)
        ci = jnp.dot((p * le1_ref[...]).astype(_BF16),
                     ew_ref[pl.ds(j * BE, BE), :],
                     preferred_element_type=_F32)
        eacc_ref[pl.ds(i * BE, BE), :] += ci

        @pl.when(i != j)
        def _():
            pt = jax.lax.transpose(p.astype(_BF16), (1, 0))
            cj = jnp.dot((pt * le2_ref[...]).astype(_BF16),
                         ew_ref[pl.ds(i * BE, BE), :],
                         preferred_element_type=_F32)
            eacc_ref[pl.ds(j * BE, BE), :] += cj

    @pl.when(t == T_STEPS - 1)
    def _():
        nodes_ref[...] = nacc_ref[...] + x_ref[...] + bn_ref[...]
        edges_ref[...] = eacc_ref[...] + e_ref[...] + be_ref[...]


def kernel(x, node_laplacian, edge_laplacian, incidence, e, W_n, W_e,
           p_node, p_edge, b_n, b_e):
    bn2 = b_n.reshape(1, D_NODE)
    be2 = b_e.reshape(1, D_EDGE)
    inc_bf = incidence.astype(_BF16)
    tab = jnp.asarray(_build_table())

    full = lambda t, tab_ref: (0, 0)

    grid_spec = pltpu.PrefetchScalarGridSpec(
        num_scalar_prefetch=1,
        grid=(T_STEPS,),
        in_specs=[
            pl.BlockSpec((N, E), full),
            pl.BlockSpec((BN, BN), lambda t, s: (s[0, t], s[1, t])),
            pl.BlockSpec((BN, BN), lambda t, s: (s[2, t], s[3, t])),
            pl.BlockSpec((BE, BE), lambda t, s: (s[4, t], s[5, t])),
            pl.BlockSpec((BE, BE), lambda t, s: (s[6, t], s[7, t])),
            pl.BlockSpec((N, D_NODE), full),
            pl.BlockSpec((E, D_EDGE), full),
            pl.BlockSpec((D_NODE, D_NODE), full),
            pl.BlockSpec((D_EDGE, D_EDGE), full),
            pl.BlockSpec((D_EDGE, 1), full),
            pl.BlockSpec((D_NODE, 1), full),
            pl.BlockSpec((1, D_NODE), full),
            pl.BlockSpec((1, D_EDGE), full),
        ],
        out_specs=[
            pl.BlockSpec((N, D_NODE), full),
            pl.BlockSpec((E, D_EDGE), full),
        ],
        scratch_shapes=[
            pltpu.VMEM((N, E), _BF16),
            pltpu.VMEM((1, E), _BF16),
            pltpu.VMEM((N, 1), _BF16),
            pltpu.VMEM((N, D_NODE), _BF16),
            pltpu.VMEM((E, D_EDGE), _BF16),
            pltpu.VMEM((N, BE), _BF16),
            pltpu.VMEM((N, D_NODE), _F32),
            pltpu.VMEM((E, D_EDGE), _F32),
        ],
    )

    new_nodes, new_edges = pl.pallas_call(
        _mega_kernel,
        grid_spec=grid_spec,
        out_shape=[
            jax.ShapeDtypeStruct((N, D_NODE), _F32),
            jax.ShapeDtypeStruct((E, D_EDGE), _F32),
        ],
        compiler_params=pltpu.CompilerParams(
            dimension_semantics=("arbitrary",)),
    )(tab, inc_bf, node_laplacian, node_laplacian,
      edge_laplacian, edge_laplacian, x, e, W_n, W_e,
      p_node, p_edge, bn2, be2)

    return new_nodes, new_edges
